# baseline (speedup 1.0000x reference)
import functools

import jax
import jax.numpy as jnp
from jax import lax
from jax.experimental import pallas as pl
from jax.experimental.pallas import tpu as pltpu
from jax.experimental.pallas import tpu_sc as plsc

N = 10000
D = 128
E = 320000
NC, NS, LN = 2, 16, 16
CHUNK = 128
CPW = 80
EPW = CHUNK * CPW
E_PAD = NC * NS * EPW
RPS = 632
ACC_ROWS = NS * RPS
RB = 1000
GRID = N // RB

_mesh = plsc.VectorSubcoreMesh(
    core_axis_name="c", subcore_axis_name="s", num_cores=NC, num_subcores=NS
)


def _deg_body(dsts_hbm, zeros_hbm, out_hbm, dst_v, ones_v, accd, sem):
    c = lax.axis_index("c")
    s = lax.axis_index("s")
    pltpu.sync_copy(dsts_hbm.at[c, s], dst_v)

    def fill(i, carry):
        ones_v[i] = jnp.full((LN,), 1.0, jnp.float32)
        return carry

    lax.fori_loop(0, CHUNK, fill, 0)
    base = s * RPS
    pltpu.sync_copy(zeros_hbm, accd.at[pl.ds(base, RPS)])
    plsc.subcore_barrier()

    def step(i, carry):
        pltpu.sync_copy(ones_v, accd.at[dst_v.at[i]], add=True)
        return carry

    lax.fori_loop(0, CPW, step, 0)
    plsc.subcore_barrier()
    pltpu.sync_copy(accd.at[pl.ds(base, RPS)], out_hbm.at[c, pl.ds(base, RPS)])


_deg = functools.partial(
    pl.kernel,
    out_type=jax.ShapeDtypeStruct((NC, ACC_ROWS, LN), jnp.float32),
    mesh=_mesh,
    scratch_types=[
        pltpu.VMEM((CPW, CHUNK), jnp.int32),
        pltpu.VMEM((CHUNK, LN), jnp.float32),
        pltpu.VMEM_SHARED((ACC_ROWS, LN), jnp.float32),
        pltpu.SemaphoreType.DMA,
    ],
)(_deg_body)


def _unpack_chunk(vpk, j, out32):
    j2 = j // 2
    off = (j % 2) * (CHUNK // 2)
    mask = jnp.full((LN,), 0xFFFF, jnp.int32)
    sh16 = jnp.full((LN,), 16, jnp.int32)
    for q in range(CHUNK // 32):
        w = vpk[j2, pl.ds(off + q * 16, 16)]
        out32[0, pl.ds(q * 32, 16)] = w & mask
        out32[0, pl.ds(q * 32 + 16, 16)] = jax.lax.shift_right_logical(w, sh16)


def _msg_body(
    u_hbm, srcs_hbm, dsts_hbm, zeros_hbm, out_hbm,
    src16_v, dst16_v, s_a, s_b, dbuf, rows0, rows1, acc, sem0, sem1,
):
    c = lax.axis_index("c")
    s = lax.axis_index("s")
    pltpu.sync_copy(srcs_hbm.at[c, s], src16_v)
    pltpu.sync_copy(dsts_hbm.at[c, s], dst16_v)
    base = s * RPS
    pltpu.sync_copy(zeros_hbm, acc.at[pl.ds(base, RPS)])
    plsc.subcore_barrier()

    _unpack_chunk(src16_v, 0, s_a)
    pltpu.async_copy(u_hbm.at[s_a.at[0]], rows0, sem0)

    def step(j, carry):
        def do(rows_cur, sem_cur, rows_nxt, sem_nxt, sbuf_cur, sbuf_nxt):
            @pl.when(j + 1 < CPW)
            def _():
                _unpack_chunk(src16_v, j + 1, sbuf_nxt)
                pltpu.async_copy(u_hbm.at[sbuf_nxt.at[0]], rows_nxt, sem_nxt)

            pltpu.make_async_copy(u_hbm.at[sbuf_cur.at[0]], rows_cur, sem_cur).wait()
            _unpack_chunk(dst16_v, j, dbuf)
            pltpu.sync_copy(rows_cur, acc.at[dbuf.at[0]], add=True)

        @pl.when(j % 2 == 0)
        def _():
            do(rows0, sem0, rows1, sem1, s_a, s_b)

        @pl.when(j % 2 == 1)
        def _():
            do(rows1, sem1, rows0, sem0, s_b, s_a)

        return carry

    lax.fori_loop(0, CPW, step, 0)
    plsc.subcore_barrier()
    pltpu.sync_copy(acc.at[pl.ds(base, RPS)], out_hbm.at[c, pl.ds(base, RPS)])


_msg = functools.partial(
    pl.kernel,
    out_type=jax.ShapeDtypeStruct((NC, ACC_ROWS, D), jnp.float32),
    mesh=_mesh,
    scratch_types=[
        pltpu.VMEM((CPW // 2, CHUNK), jnp.int32),
        pltpu.VMEM((CPW // 2, CHUNK), jnp.int32),
        pltpu.VMEM((8, CHUNK), jnp.int32),
        pltpu.VMEM((8, CHUNK), jnp.int32),
        pltpu.VMEM((8, CHUNK), jnp.int32),
        pltpu.VMEM((CHUNK, D), jnp.float32),
        pltpu.VMEM((CHUNK, D), jnp.float32),
        pltpu.VMEM_SHARED((ACC_ROWS, D), jnp.float32),
        pltpu.SemaphoreType.DMA,
        pltpu.SemaphoreType.DMA,
    ],
)(_msg_body)


def _tc_first_body(deg_ref, x_ref, w_ref, u_ref, dinv_ref):
    deg = jnp.sum(deg_ref[0] + deg_ref[1], axis=1, keepdims=True) * (1.0 / LN)
    dinv = jax.lax.rsqrt(deg + 1.0)
    dinv = jnp.broadcast_to(dinv, (RB, D))
    z = jnp.dot(x_ref[...], w_ref[...], preferred_element_type=jnp.float32)
    u_ref[...] = z * dinv
    dinv_ref[...] = dinv


def _tc_first(degs, x, w):
    return pl.pallas_call(
        _tc_first_body,
        grid=(GRID,),
        in_specs=[
            pl.BlockSpec((NC, RB, LN), lambda i: (0, i, 0)),
            pl.BlockSpec((RB, D), lambda i: (i, 0)),
            pl.BlockSpec((D, D), lambda i: (0, 0)),
        ],
        out_specs=[
            pl.BlockSpec((RB, D), lambda i: (i, 0)),
            pl.BlockSpec((RB, D), lambda i: (i, 0)),
        ],
        out_shape=[
            jax.ShapeDtypeStruct((N, D), jnp.float32),
            jax.ShapeDtypeStruct((N, D), jnp.float32),
        ],
    )(degs, x, w)


def _tc_mid_body(p_ref, u_ref, dinv_ref, b_ref, w_ref, o_ref):
    t = dinv_ref[...] * (p_ref[0] + p_ref[1] + u_ref[...]) + b_ref[...][0:1]
    t = jnp.maximum(t, 0.0)
    o_ref[...] = (
        jnp.dot(t, w_ref[...], preferred_element_type=jnp.float32) * dinv_ref[...]
    )


def _tc_mid(p, u, dinv2d, b2d, w):
    return pl.pallas_call(
        _tc_mid_body,
        grid=(GRID,),
        in_specs=[
            pl.BlockSpec((NC, RB, D), lambda i: (0, i, 0)),
            pl.BlockSpec((RB, D), lambda i: (i, 0)),
            pl.BlockSpec((RB, D), lambda i: (i, 0)),
            pl.BlockSpec((8, D), lambda i: (0, 0)),
            pl.BlockSpec((D, D), lambda i: (0, 0)),
        ],
        out_specs=pl.BlockSpec((RB, D), lambda i: (i, 0)),
        out_shape=jax.ShapeDtypeStruct((N, D), jnp.float32),
    )(p, u, dinv2d, b2d, w)


def _tc_final_body(p_ref, u_ref, dinv_ref, b_ref, o_ref):
    o_ref[...] = (
        dinv_ref[...] * (p_ref[0] + p_ref[1] + u_ref[...]) + b_ref[...][0:1]
    )


def _tc_final(p, u, dinv2d, b2d):
    return pl.pallas_call(
        _tc_final_body,
        grid=(GRID,),
        in_specs=[
            pl.BlockSpec((NC, RB, D), lambda i: (0, i, 0)),
            pl.BlockSpec((RB, D), lambda i: (i, 0)),
            pl.BlockSpec((RB, D), lambda i: (i, 0)),
            pl.BlockSpec((8, D), lambda i: (0, 0)),
        ],
        out_specs=pl.BlockSpec((RB, D), lambda i: (i, 0)),
        out_shape=jax.ShapeDtypeStruct((N, D), jnp.float32),
    )(p, u, dinv2d, b2d)


def _b2d(b):
    return jnp.broadcast_to(b.reshape(1, D), (8, D))


def kernel(x, edge_index, W0, b0, W1, b1, W2, b2):
    src = edge_index[0].astype(jnp.int32)
    dst = edge_index[1].astype(jnp.int32)
    pad = E_PAD - E
    pad_ar = jnp.arange(pad, dtype=jnp.int32)
    srcs = jnp.concatenate([src, pad_ar % N])
    dsts = jnp.concatenate([dst, N + pad_ar % (ACC_ROWS - N)])
    dsts32 = dsts.reshape(NC, NS, CPW, CHUNK)

    def _pack(e):
        e2 = e.reshape(-1, 2)
        return (e2[:, 0] | (e2[:, 1] << 16)).reshape(NC, NS, CPW // 2, CHUNK)

    srcs = _pack(srcs)
    dsts = _pack(dsts)
    zeros_d = jnp.zeros((RPS, LN), jnp.float32)
    zeros_m = jnp.zeros((RPS, D), jnp.float32)

    degs = _deg(dsts32, zeros_d)
    u0, dinv2d = _tc_first(degs, x, W0)
    p = _msg(u0, srcs, dsts, zeros_m)
    u1 = _tc_mid(p, u0, dinv2d, _b2d(b0), W1)
    p = _msg(u1, srcs, dsts, zeros_m)
    u2 = _tc_mid(p, u1, dinv2d, _b2d(b1), W2)
    p = _msg(u2, srcs, dsts, zeros_m)
    return _tc_final(p, u2, dinv2d, _b2d(b2))


# speedup vs baseline: 1.0690x; 1.0690x over previous
import functools

import jax
import jax.numpy as jnp
from jax import lax
from jax.experimental import pallas as pl
from jax.experimental.pallas import tpu as pltpu
from jax.experimental.pallas import tpu_sc as plsc

N = 10000
D = 128
E = 320000
NC, NS, LN = 2, 16, 16
CHUNK = 128
CPW = 80
EPW = CHUNK * CPW
E_PAD = NC * NS * EPW
RPS = 632
ACC_ROWS = NS * RPS
RB = 1000
GRID = N // RB

_mesh = plsc.VectorSubcoreMesh(
    core_axis_name="c", subcore_axis_name="s", num_cores=NC, num_subcores=NS
)


def _deg_body(dsts_hbm, zeros_hbm, out_hbm, dst_v, ones_v, accd, sem):
    c = lax.axis_index("c")
    s = lax.axis_index("s")
    pltpu.sync_copy(dsts_hbm.at[c, s], dst_v)

    def fill(i, carry):
        ones_v[i] = jnp.full((LN,), 1.0, jnp.float32)
        return carry

    lax.fori_loop(0, CHUNK, fill, 0)
    base = s * RPS
    pltpu.sync_copy(zeros_hbm, accd.at[pl.ds(base, RPS)])
    plsc.subcore_barrier()

    def step(i, carry):
        pltpu.sync_copy(ones_v, accd.at[dst_v.at[i]], add=True)
        return carry

    lax.fori_loop(0, CPW, step, 0)
    plsc.subcore_barrier()
    pltpu.sync_copy(accd.at[pl.ds(base, RPS)], out_hbm.at[c, pl.ds(base, RPS)])


_deg = functools.partial(
    pl.kernel,
    out_type=jax.ShapeDtypeStruct((NC, ACC_ROWS, LN), jnp.float32),
    mesh=_mesh,
    scratch_types=[
        pltpu.VMEM((CPW, CHUNK), jnp.int32),
        pltpu.VMEM((CHUNK, LN), jnp.float32),
        pltpu.VMEM_SHARED((ACC_ROWS, LN), jnp.float32),
        pltpu.SemaphoreType.DMA,
    ],
)(_deg_body)


def _msg_body(
    u_hbm, srcs_hbm, dsts_hbm, zeros_hbm, out_hbm,
    src_v, dst_v, rows, acc, sem,
):
    c = lax.axis_index("c")
    s = lax.axis_index("s")
    pltpu.sync_copy(srcs_hbm.at[c, s], src_v)
    pltpu.sync_copy(dsts_hbm.at[c, s], dst_v)
    base = s * RPS
    pltpu.sync_copy(zeros_hbm, acc.at[pl.ds(base, RPS)])
    plsc.subcore_barrier()

    def step(j, carry):
        pltpu.async_copy(u_hbm.at[src_v.at[j]], rows, sem).wait()
        pltpu.sync_copy(rows, acc.at[dst_v.at[j]], add=True)
        return carry

    lax.fori_loop(0, CPW, step, 0)
    plsc.subcore_barrier()
    pltpu.sync_copy(acc.at[pl.ds(base, RPS)], out_hbm.at[c, pl.ds(base, RPS)])


_msg = functools.partial(
    pl.kernel,
    out_type=jax.ShapeDtypeStruct((NC, ACC_ROWS, D), jnp.float32),
    mesh=_mesh,
    scratch_types=[
        pltpu.VMEM((CPW, CHUNK), jnp.int32),
        pltpu.VMEM((CPW, CHUNK), jnp.int32),
        pltpu.VMEM((CHUNK, D), jnp.float32),
        pltpu.VMEM_SHARED((ACC_ROWS, D), jnp.float32),
        pltpu.SemaphoreType.DMA,
    ],
)(_msg_body)


def _tc_first_body(deg_ref, x_ref, w_ref, u_ref, dinv_ref):
    deg = jnp.sum(deg_ref[0] + deg_ref[1], axis=1, keepdims=True) * (1.0 / LN)
    dinv = jax.lax.rsqrt(deg + 1.0)
    dinv = jnp.broadcast_to(dinv, (RB, D))
    z = jnp.dot(x_ref[...], w_ref[...], preferred_element_type=jnp.float32)
    u_ref[...] = z * dinv
    dinv_ref[...] = dinv


def _tc_first(degs, x, w):
    return pl.pallas_call(
        _tc_first_body,
        grid=(GRID,),
        in_specs=[
            pl.BlockSpec((NC, RB, LN), lambda i: (0, i, 0)),
            pl.BlockSpec((RB, D), lambda i: (i, 0)),
            pl.BlockSpec((D, D), lambda i: (0, 0)),
        ],
        out_specs=[
            pl.BlockSpec((RB, D), lambda i: (i, 0)),
            pl.BlockSpec((RB, D), lambda i: (i, 0)),
        ],
        out_shape=[
            jax.ShapeDtypeStruct((N, D), jnp.float32),
            jax.ShapeDtypeStruct((N, D), jnp.float32),
        ],
    )(degs, x, w)


def _tc_mid_body(p_ref, u_ref, dinv_ref, b_ref, w_ref, o_ref):
    t = dinv_ref[...] * (p_ref[0] + p_ref[1] + u_ref[...]) + b_ref[...][0:1]
    t = jnp.maximum(t, 0.0)
    o_ref[...] = (
        jnp.dot(t, w_ref[...], preferred_element_type=jnp.float32) * dinv_ref[...]
    )


def _tc_mid(p, u, dinv2d, b2d, w):
    return pl.pallas_call(
        _tc_mid_body,
        grid=(GRID,),
        in_specs=[
            pl.BlockSpec((NC, RB, D), lambda i: (0, i, 0)),
            pl.BlockSpec((RB, D), lambda i: (i, 0)),
            pl.BlockSpec((RB, D), lambda i: (i, 0)),
            pl.BlockSpec((8, D), lambda i: (0, 0)),
            pl.BlockSpec((D, D), lambda i: (0, 0)),
        ],
        out_specs=pl.BlockSpec((RB, D), lambda i: (i, 0)),
        out_shape=jax.ShapeDtypeStruct((N, D), jnp.float32),
    )(p, u, dinv2d, b2d, w)


def _tc_final_body(p_ref, u_ref, dinv_ref, b_ref, o_ref):
    o_ref[...] = (
        dinv_ref[...] * (p_ref[0] + p_ref[1] + u_ref[...]) + b_ref[...][0:1]
    )


def _tc_final(p, u, dinv2d, b2d):
    return pl.pallas_call(
        _tc_final_body,
        grid=(GRID,),
        in_specs=[
            pl.BlockSpec((NC, RB, D), lambda i: (0, i, 0)),
            pl.BlockSpec((RB, D), lambda i: (i, 0)),
            pl.BlockSpec((RB, D), lambda i: (i, 0)),
            pl.BlockSpec((8, D), lambda i: (0, 0)),
        ],
        out_specs=pl.BlockSpec((RB, D), lambda i: (i, 0)),
        out_shape=jax.ShapeDtypeStruct((N, D), jnp.float32),
    )(p, u, dinv2d, b2d)


def _b2d(b):
    return jnp.broadcast_to(b.reshape(1, D), (8, D))


def kernel(x, edge_index, W0, b0, W1, b1, W2, b2):
    src = edge_index[0].astype(jnp.int32)
    dst = edge_index[1].astype(jnp.int32)
    pad = E_PAD - E
    pad_ar = jnp.arange(pad, dtype=jnp.int32)
    srcs = jnp.concatenate([src, pad_ar % N])
    dsts = jnp.concatenate([dst, N + pad_ar % (ACC_ROWS - N)])
    srcs = srcs.reshape(NC, NS, CPW, CHUNK)
    dsts32 = dsts.reshape(NC, NS, CPW, CHUNK)
    dsts = dsts32
    zeros_d = jnp.zeros((RPS, LN), jnp.float32)
    zeros_m = jnp.zeros((RPS, D), jnp.float32)

    degs = _deg(dsts32, zeros_d)
    u0, dinv2d = _tc_first(degs, x, W0)
    p = _msg(u0, srcs, dsts, zeros_m)
    u1 = _tc_mid(p, u0, dinv2d, _b2d(b0), W1)
    p = _msg(u1, srcs, dsts, zeros_m)
    u2 = _tc_mid(p, u1, dinv2d, _b2d(b1), W2)
    p = _msg(u2, srcs, dsts, zeros_m)
    return _tc_final(p, u2, dinv2d, _b2d(b2))
